# baseline (speedup 1.0000x reference)
import functools

import jax
import jax.numpy as jnp
from jax import lax
from jax.experimental import pallas as pl
from jax.experimental.pallas import tpu as pltpu
from jax.experimental.pallas import tpu_sc as plsc

_NT = 32768
_H = 768
_NE = 64
_BT = 4096

_NW = 32
_TPW = _NT // _NW
_G = _TPW // 16


def _matmul_body(x_ref, w_ref, lg_ref):
    lg_ref[...] = jax.lax.dot_general(
        x_ref[...], w_ref[...],
        dimension_numbers=(((1,), (1,)), ((), ())),
        preferred_element_type=jnp.float32)


def _tc_logits(x, W):
    return pl.pallas_call(
        _matmul_body,
        grid=(_NT // _BT,),
        in_specs=[
            pl.BlockSpec((_BT, _H), lambda i: (i, 0)),
            pl.BlockSpec((_NE, _H), lambda i: (0, 0)),
        ],
        out_specs=pl.BlockSpec((_BT, _NE), lambda i: (i, 0)),
        out_shape=jax.ShapeDtypeStruct((_NT, _NE), jnp.float32),
        compiler_params=pltpu.CompilerParams(
            dimension_semantics=("arbitrary",)),
    )(x, W)


def _route_body(lg_hbm, w_hbm, e_hbm, buf, wbuf, ebuf):
    wid = lax.axis_index("s") * 2 + lax.axis_index("c")
    base = wid * _TPW
    pltpu.sync_copy(lg_hbm.at[pl.ds(base * _NE, _TPW * _NE)], buf)
    lane = lax.iota(jnp.int32, 16)

    def group(g, carry):
        rows = g * 16 + lane
        zero = jnp.zeros((16,), jnp.int32)
        flat = rows * _NE
        m1 = plsc.load_gather(buf, [flat])
        i1 = zero
        m2 = jnp.full((16,), -jnp.inf, jnp.float32)
        i2 = zero
        for e in range(1, _NE):
            ev = jnp.full((16,), e, jnp.int32)
            v = plsc.load_gather(buf, [flat + e])
            c1 = v > m1
            c2 = v > m2
            m2 = jnp.where(c1, m1, jnp.where(c2, v, m2))
            i2 = jnp.where(c1, i1, jnp.where(c2, ev, i2))
            m1 = jnp.where(c1, v, m1)
            i1 = jnp.where(c1, ev, i1)
        t = jnp.exp(m2 - m1)
        d = 1.0 + t
        slots = rows * 2
        plsc.store_scatter(wbuf, [slots], 1.0 / d)
        plsc.store_scatter(wbuf, [slots + 1], t / d)
        plsc.store_scatter(ebuf, [slots], i1)
        plsc.store_scatter(ebuf, [slots + 1], i2)
        return carry

    lax.fori_loop(0, _G, group, 0)
    pltpu.sync_copy(wbuf, w_hbm.at[pl.ds(base * 2, _TPW * 2)])
    pltpu.sync_copy(ebuf, e_hbm.at[pl.ds(base * 2, _TPW * 2)])


_route = pl.kernel(
    _route_body,
    out_type=[
        jax.ShapeDtypeStruct((_NT * 2,), jnp.float32),
        jax.ShapeDtypeStruct((_NT * 2,), jnp.int32),
    ],
    mesh=plsc.VectorSubcoreMesh(core_axis_name="c", subcore_axis_name="s"),
    compiler_params=pltpu.CompilerParams(needs_layout_passes=False),
    scratch_types=[
        pltpu.VMEM((_TPW * _NE,), jnp.float32),
        pltpu.VMEM((_TPW * 2,), jnp.float32),
        pltpu.VMEM((_TPW * 2,), jnp.int32),
    ],
)


def kernel(x, W):
    logits = _tc_logits(x, W)
    wflat, eflat = _route(logits.reshape(-1))
    return (wflat.reshape(_NT, 2), eflat.reshape(_NT, 2))


# speedup vs baseline: 1.3065x; 1.3065x over previous
import jax
import jax.numpy as jnp
from jax import lax
from jax.experimental import pallas as pl
from jax.experimental.pallas import tpu as pltpu
from jax.experimental.pallas import tpu_sc as plsc

_NT = 32768
_H = 768
_NE = 64
_BT = 4096

_NW = 32
_TPW = _NT // _NW
_IL = 4
_G = _TPW // (16 * _IL)


def _matmul_body(x_ref, w_ref, lg_ref):
    lg_ref[...] = jax.lax.dot_general(
        w_ref[...], x_ref[...],
        dimension_numbers=(((1,), (1,)), ((), ())),
        preferred_element_type=jnp.float32)


def _tc_logits_t(x, W):
    return pl.pallas_call(
        _matmul_body,
        grid=(_NT // _BT,),
        in_specs=[
            pl.BlockSpec((_BT, _H), lambda i: (i, 0)),
            pl.BlockSpec((_NE, _H), lambda i: (0, 0)),
        ],
        out_specs=pl.BlockSpec((_NE, _BT), lambda i: (0, i)),
        out_shape=jax.ShapeDtypeStruct((_NE, _NT), jnp.float32),
        compiler_params=pltpu.CompilerParams(
            dimension_semantics=("arbitrary",)),
    )(x, W)


def _route_body(lg_hbm, w_hbm, e_hbm, buf, wbuf, ebuf):
    wid = lax.axis_index("s") * 2 + lax.axis_index("c")
    base = wid * _TPW
    pltpu.sync_copy(lg_hbm.at[:, pl.ds(base, _TPW)], buf)
    lane = lax.iota(jnp.int32, 16)

    def block(g, carry):
        off = g * (16 * _IL)
        m1 = [buf[0, pl.ds(off + 16 * j, 16)] for j in range(_IL)]
        i1 = [jnp.zeros((16,), jnp.int32) for _ in range(_IL)]
        m2 = [jnp.full((16,), -jnp.inf, jnp.float32) for _ in range(_IL)]
        i2 = [jnp.zeros((16,), jnp.int32) for _ in range(_IL)]
        for e in range(1, _NE):
            ev = jnp.full((16,), e, jnp.int32)
            for j in range(_IL):
                v = buf[e, pl.ds(off + 16 * j, 16)]
                c1 = v > m1[j]
                c2 = v > m2[j]
                m2[j] = jnp.where(c1, m1[j], jnp.where(c2, v, m2[j]))
                i2[j] = jnp.where(c1, i1[j], jnp.where(c2, ev, i2[j]))
                m1[j] = jnp.where(c1, v, m1[j])
                i1[j] = jnp.where(c1, ev, i1[j])
        for j in range(_IL):
            t = jnp.exp(m2[j] - m1[j])
            d = 1.0 + t
            slots = (off + 16 * j + lane) * 2
            plsc.store_scatter(wbuf, [slots], 1.0 / d)
            plsc.store_scatter(wbuf, [slots + 1], t / d)
            plsc.store_scatter(ebuf, [slots], i1[j])
            plsc.store_scatter(ebuf, [slots + 1], i2[j])
        return carry

    lax.fori_loop(0, _G, block, 0)
    pltpu.sync_copy(wbuf, w_hbm.at[pl.ds(base * 2, _TPW * 2)])
    pltpu.sync_copy(ebuf, e_hbm.at[pl.ds(base * 2, _TPW * 2)])


_route = pl.kernel(
    _route_body,
    out_type=[
        jax.ShapeDtypeStruct((_NT * 2,), jnp.float32),
        jax.ShapeDtypeStruct((_NT * 2,), jnp.int32),
    ],
    mesh=plsc.VectorSubcoreMesh(core_axis_name="c", subcore_axis_name="s"),
    compiler_params=pltpu.CompilerParams(needs_layout_passes=False),
    scratch_types=[
        pltpu.VMEM((_NE, _TPW), jnp.float32),
        pltpu.VMEM((_TPW * 2,), jnp.float32),
        pltpu.VMEM((_TPW * 2,), jnp.int32),
    ],
)


def kernel(x, W):
    logits_t = _tc_logits_t(x, W)
    wflat, eflat = _route(logits_t)
    return (wflat.reshape(_NT, 2), eflat.reshape(_NT, 2))


# speedup vs baseline: 4.0425x; 3.0941x over previous
import jax
import jax.numpy as jnp
from jax.experimental import pallas as pl
from jax.experimental.pallas import tpu as pltpu

_NT = 32768
_H = 768
_NE = 64
_BT = 4096


def _body(x_ref, w_ref, w1_ref, w2_ref, i1_ref, i2_ref):
    logits = jax.lax.dot_general(
        w_ref[...], x_ref[...],
        dimension_numbers=(((1,), (1,)), ((), ())),
        preferred_element_type=jnp.float32)
    e_ids = jax.lax.broadcasted_iota(jnp.int32, logits.shape, 0)
    m1 = jnp.max(logits, axis=0, keepdims=True)
    i1 = jnp.min(jnp.where(logits == m1, e_ids, _NE), axis=0, keepdims=True)
    masked = jnp.where(e_ids == i1, -jnp.inf, logits)
    m2 = jnp.max(masked, axis=0, keepdims=True)
    i2 = jnp.min(jnp.where(masked == m2, e_ids, _NE), axis=0, keepdims=True)
    t = jnp.exp(m2 - m1)
    d = 1.0 + t
    w1_ref[...] = 1.0 / d
    w2_ref[...] = t / d
    i1_ref[...] = i1
    i2_ref[...] = i2


def kernel(x, W):
    w1, w2, i1, i2 = pl.pallas_call(
        _body,
        grid=(_NT // _BT,),
        in_specs=[
            pl.BlockSpec((_BT, _H), lambda i: (i, 0)),
            pl.BlockSpec((_NE, _H), lambda i: (0, 0)),
        ],
        out_specs=[
            pl.BlockSpec((1, _BT), lambda i: (0, i)),
            pl.BlockSpec((1, _BT), lambda i: (0, i)),
            pl.BlockSpec((1, _BT), lambda i: (0, i)),
            pl.BlockSpec((1, _BT), lambda i: (0, i)),
        ],
        out_shape=[
            jax.ShapeDtypeStruct((1, _NT), jnp.float32),
            jax.ShapeDtypeStruct((1, _NT), jnp.float32),
            jax.ShapeDtypeStruct((1, _NT), jnp.int32),
            jax.ShapeDtypeStruct((1, _NT), jnp.int32),
        ],
        compiler_params=pltpu.CompilerParams(
            dimension_semantics=("arbitrary",)),
    )(x, W)
    rw = jnp.stack([w1[0], w2[0]], axis=-1)
    se = jnp.stack([i1[0], i2[0]], axis=-1)
    return (rw, se)
